# baseline (speedup 1.0000x reference)
import jax
import jax.numpy as jnp
from jax.experimental import pallas as pl
from jax.experimental.pallas import tpu as pltpu

HIDDEN = 2048
N_EXPERTS = 64
BLOCK_K = 256


def _router_kernel(hs_ref, w_ref, out_ref):
    i = pl.program_id(0)
    acc = jax.lax.dot_general(
        hs_ref[...],
        w_ref[...],
        dimension_numbers=(((1,), (1,)), ((), ())),
        preferred_element_type=jnp.float32,
    )

    @pl.when(i == 0)
    def _():
        out_ref[...] = acc

    @pl.when(i > 0)
    def _():
        out_ref[...] = out_ref[...] + acc


def kernel(hidden_states, weight):
    hs = hidden_states.reshape(-1, HIDDEN).astype(jnp.float32)
    w = weight.astype(jnp.float32)
    m = hs.shape[0]
    return pl.pallas_call(
        _router_kernel,
        grid=(HIDDEN // BLOCK_K,),
        in_specs=[
            pl.BlockSpec((m, BLOCK_K), lambda i: (0, i)),
            pl.BlockSpec((N_EXPERTS, BLOCK_K), lambda i: (0, i)),
        ],
        out_specs=pl.BlockSpec((m, N_EXPERTS), lambda i: (0, 0)),
        out_shape=jax.ShapeDtypeStruct((m, N_EXPERTS), jnp.float32),
    )(hs, w)


# speedup vs baseline: 1.0560x; 1.0560x over previous
import jax
import jax.numpy as jnp
from jax.experimental import pallas as pl
from jax.experimental.pallas import tpu as pltpu

HIDDEN = 2048
N_EXPERTS = 64
BLOCK_M = 1024


def _router_kernel(hs_ref, w_ref, out_ref):
    out_ref[...] = jax.lax.dot_general(
        hs_ref[...],
        w_ref[...],
        dimension_numbers=(((1,), (1,)), ((), ())),
        preferred_element_type=jnp.float32,
    )


def kernel(hidden_states, weight):
    hs = hidden_states.reshape(-1, HIDDEN).astype(jnp.float32)
    w = weight.astype(jnp.float32)
    m = hs.shape[0]
    return pl.pallas_call(
        _router_kernel,
        grid=(m // BLOCK_M,),
        in_specs=[
            pl.BlockSpec((BLOCK_M, HIDDEN), lambda i: (i, 0)),
            pl.BlockSpec(memory_space=pltpu.VMEM),
        ],
        out_specs=pl.BlockSpec((BLOCK_M, N_EXPERTS), lambda i: (i, 0)),
        out_shape=jax.ShapeDtypeStruct((m, N_EXPERTS), jnp.float32),
    )(hs, w)
